# Initial kernel scaffold; baseline (speedup 1.0000x reference)
import jax
import jax.numpy as jnp
from jax.experimental import pallas as pl


def kernel(features, labels):
    raise NotImplementedError("write your pallas kernel here")


# speedup vs baseline: 3.0378x; 3.0378x over previous
import jax
import jax.numpy as jnp
from jax import lax
from jax.experimental import pallas as pl
from jax.experimental.pallas import tpu as pltpu
from jax.experimental.pallas import tpu_sc as plsc

_K = 32
_D = 32
_N = 32768
_NT = 16
_C = _N // _NT
_G = _C // 16
_DELTA_VAR = 0.5
_DELTA_DIST = 1.5
_ALPHA = 0.1
_BETA = 1.0
_GAMMA = 0.001
_EPS = 1e-12


def _vsqrt(x):
    i = plsc.bitcast(x, jnp.int32)
    i = 0x5F3759DF - (i >> 1)
    r = plsc.bitcast(i, jnp.float32)
    for _ in range(3):
        r = r * (1.5 - 0.5 * x * r * r)
    return x * r


def _splat(s):
    return jnp.zeros((16,), jnp.float32) + s


def _sc_body(f_hbm, l_hbm, out_hbm,
             fv, lv, accv, cnt2d, h2acc, meansv, scntv, mtv,
             pbuf, pcbuf, ph2buf, outv,
             psums_sp, pcnts_sp, means_sp, cnts_sp, ph2_sp):
    s = lax.axis_index("s")
    c = lax.axis_index("c")
    iota16 = lax.iota(jnp.int32, 16)
    ones16 = jnp.ones((16,), jnp.float32)
    zeros16 = jnp.zeros((16,), jnp.float32)

    pltpu.sync_copy(f_hbm.at[pl.ds(s * _C * _D, _C * _D)], fv)
    pltpu.sync_copy(l_hbm.at[pl.ds(s * _C, _C)], lv)

    @pl.loop(0, _K)
    def _(k):
        accv[pl.ds(k * _D, 16)] = zeros16
        accv[pl.ds(k * _D + 16, 16)] = zeros16
        cnt2d[pl.ds(k * 16, 16)] = zeros16
        h2acc[pl.ds(k * 16, 16)] = zeros16

    h2acc[pl.ds(_K * 16, 16)] = zeros16

    @pl.loop(0, _G)
    def _(g):
        b = g * 16
        lblv = lv[pl.ds(b, 16)]
        for i in range(16):
            lbl = lblv[i]
            plsc.addupdate(accv.at[pl.ds(lbl * _D, 16)], fv[pl.ds((b + i) * _D, 16)])
            plsc.addupdate(accv.at[pl.ds(lbl * _D + 16, 16)],
                           fv[pl.ds((b + i) * _D + 16, 16)])
            plsc.addupdate(cnt2d.at[pl.ds(lbl * 16, 16)],
                           jnp.where(iota16 == i, 1.0, 0.0))

    pltpu.sync_copy(accv, psums_sp.at[s])
    pltpu.sync_copy(cnt2d, pcnts_sp.at[s])
    plsc.subcore_barrier()

    @pl.loop(0, _NT)
    def _(tt):
        pltpu.sync_copy(psums_sp.at[tt, pl.ds(2 * s * _D, 2 * _D)], pbuf.at[tt])
        pltpu.sync_copy(pcnts_sp.at[tt, pl.ds(2 * s * 16, 32)], pcbuf.at[tt])

    for ki in range(2):
        k = 2 * s + ki

        def _red(tt, carry, ki=ki):
            s0, s1, cv = carry
            return (s0 + pbuf[tt, ki * _D:ki * _D + 16],
                    s1 + pbuf[tt, ki * _D + 16:ki * _D + 32],
                    cv + pcbuf[tt, ki * 16:ki * 16 + 16])

        s0, s1, cv = pl.loop(0, _NT, init_carry=(zeros16, zeros16, zeros16))(_red)
        safe = jnp.maximum(jnp.sum(cv), 1.0)
        safev = _splat(safe)
        meansv[pl.ds(k * _D, 16)] = s0 / safev
        meansv[pl.ds(k * _D + 16, 16)] = s1 / safev
        scntv[pl.ds(k * 16, 16)] = safev

    pltpu.sync_copy(meansv.at[pl.ds(2 * s * _D, 2 * _D)],
                    means_sp.at[pl.ds(2 * s * _D, 2 * _D)])
    pltpu.sync_copy(scntv.at[pl.ds(2 * s * 16, 32)], cnts_sp.at[pl.ds(2 * s * 16, 32)])
    plsc.subcore_barrier()

    pltpu.sync_copy(means_sp, meansv)
    pltpu.sync_copy(cnts_sp, scntv)

    @pl.loop(0, _G)
    def _(g):
        b = g * 16
        lblv = lv[pl.ds(b, 16)]
        pbase = (iota16 + b) * _D
        mbase = lblv * _D
        d2 = zeros16
        for d in range(_D):
            xd = plsc.load_gather(fv, [pbase + d])
            md = plsc.load_gather(meansv, [mbase + d])
            t = xd - md
            d2 = d2 + t * t
        dist = _vsqrt(jnp.maximum(d2, _EPS))
        h = jnp.maximum(dist - _DELTA_VAR, 0.0)
        h2 = h * h
        for i in range(16):
            lbl = lblv[i]
            plsc.addupdate(h2acc.at[pl.ds(lbl * 16, 16)],
                           jnp.where(iota16 == i, h2, 0.0))

    cbaseA = iota16 * _D
    cbaseB = (iota16 + 16) * _D
    r0 = zeros16
    r1 = zeros16
    for d in range(_D):
        colA = plsc.load_gather(meansv, [cbaseA + d])
        colB = plsc.load_gather(meansv, [cbaseB + d])
        mtv[d, 0:16] = colA
        mtv[d, 16:32] = colB
        r0 = r0 + colA * colA
        r1 = r1 + colB * colB
    regn0 = _vsqrt(jnp.maximum(r0, _EPS))
    regn1 = _vsqrt(jnp.maximum(r1, _EPS))
    reg_sum = jnp.sum(regn0) + jnp.sum(regn1)

    dist_part = jnp.float32(0.0)
    for ki in range(2):
        k = 2 * s + ki
        mk0 = meansv[pl.ds(k * _D, 16)]
        mk1 = meansv[pl.ds(k * _D + 16, 16)]
        for jc in range(2):
            pd2 = zeros16
            for d in range(_D):
                mkd = mk0[d] if d < 16 else mk1[d - 16]
                t = mtv[d, jc * 16:jc * 16 + 16] - mkd
                pd2 = pd2 + t * t
            jv = iota16 + 16 * jc
            diag = jv == k
            pd = _vsqrt(jnp.where(diag, 1.0, jnp.maximum(pd2, _EPS)))
            h = jnp.maximum(2.0 * _DELTA_DIST - pd, 0.0)
            dist_part = dist_part + jnp.sum(jnp.where(diag, 0.0, h * h))

    h2acc[pl.ds(_K * 16, 16)] = jnp.where(iota16 == 0, dist_part, 0.0)
    pltpu.sync_copy(h2acc, ph2_sp.at[s])
    plsc.subcore_barrier()

    @pl.when(jnp.logical_and(s == 0, c == 0))
    def _():
        pltpu.sync_copy(ph2_sp, ph2buf)

        def _var(k, vacc):
            def _acc(tt, a16):
                return a16 + ph2buf[tt, pl.ds(k * 16, 16)]

            a16 = pl.loop(0, _NT, init_carry=zeros16)(_acc)
            return vacc + a16 / scntv[pl.ds(k * 16, 16)]

        var_vacc = pl.loop(0, _K, init_carry=zeros16)(_var)
        var_loss = jnp.sum(var_vacc) * (1.0 / _K)

        def _dist(tt, a16):
            return a16 + ph2buf[tt, pl.ds(_K * 16, 16)]

        dist_v = pl.loop(0, _NT, init_carry=zeros16)(_dist)
        dist_loss = jnp.sum(dist_v) * (1.0 / (_K * (_K - 1)))

        reg_loss = reg_sum * (1.0 / _K)
        loss = _ALPHA * var_loss + _BETA * dist_loss + _GAMMA * reg_loss

        res = jnp.where(iota16 == 0, loss,
                        jnp.where(iota16 == 1, var_loss,
                                  jnp.where(iota16 == 2, dist_loss,
                                            jnp.where(iota16 == 3, reg_loss, 0.0))))
        outv[:] = res
        pltpu.sync_copy(outv, out_hbm)


@jax.jit
def _run(features, labels):
    mesh = plsc.VectorSubcoreMesh(core_axis_name="c", subcore_axis_name="s",
                                  num_cores=2, num_subcores=16)
    f = pl.kernel(
        _sc_body,
        out_type=jax.ShapeDtypeStruct((16,), jnp.float32),
        mesh=mesh,
        compiler_params=pltpu.CompilerParams(needs_layout_passes=False),
        scratch_types=[
            pltpu.VMEM((_C * _D,), jnp.float32),
            pltpu.VMEM((_C,), jnp.int32),
            pltpu.VMEM((_K * _D,), jnp.float32),
            pltpu.VMEM((_K * 16,), jnp.float32),
            pltpu.VMEM((_K * 16 + 16,), jnp.float32),
            pltpu.VMEM((_K * _D,), jnp.float32),
            pltpu.VMEM((_K * 16,), jnp.float32),
            pltpu.VMEM((_D, _K), jnp.float32),
            pltpu.VMEM((_NT, 2 * _D), jnp.float32),
            pltpu.VMEM((_NT, 32), jnp.float32),
            pltpu.VMEM((_NT, _K * 16 + 16), jnp.float32),
            pltpu.VMEM((16,), jnp.float32),
            pltpu.VMEM_SHARED((_NT, _K * _D), jnp.float32),
            pltpu.VMEM_SHARED((_NT, _K * 16), jnp.float32),
            pltpu.VMEM_SHARED((_K * _D,), jnp.float32),
            pltpu.VMEM_SHARED((_K * 16,), jnp.float32),
            pltpu.VMEM_SHARED((_NT, _K * 16 + 16), jnp.float32),
        ],
    )
    return f(features.reshape(-1), labels)


def kernel(features, labels):
    out = _run(features, labels)
    return out[0], out[1], out[2], out[3]


# speedup vs baseline: 3.1371x; 1.0327x over previous
import jax
import jax.numpy as jnp
from jax import lax
from jax.experimental import pallas as pl
from jax.experimental.pallas import tpu as pltpu
from jax.experimental.pallas import tpu_sc as plsc

_K = 32
_D = 32
_N = 32768
_NT = 16
_C = _N // _NT
_G = _C // 16
_DELTA_VAR = 0.5
_DELTA_DIST = 1.5
_ALPHA = 0.1
_BETA = 1.0
_GAMMA = 0.001
_EPS = 1e-12


def _vsqrt(x):
    i = plsc.bitcast(x, jnp.int32)
    i = 0x5F3759DF - (i >> 1)
    r = plsc.bitcast(i, jnp.float32)
    for _ in range(3):
        r = r * (1.5 - 0.5 * x * r * r)
    return x * r


def _splat(s):
    return jnp.zeros((16,), jnp.float32) + s


def _sc_body(f_hbm, l_hbm, out_hbm,
             fv, lv, accv, cnt2d, h2acc, meansv, scntv, mtv,
             pbuf, pcbuf, ph2buf, outv,
             psums_sp, pcnts_sp, means_sp, cnts_sp, ph2_sp):
    s = lax.axis_index("s")
    c = lax.axis_index("c")
    iota16 = lax.iota(jnp.int32, 16)
    ones16 = jnp.ones((16,), jnp.float32)
    zeros16 = jnp.zeros((16,), jnp.float32)

    pltpu.sync_copy(f_hbm.at[pl.ds(s * _C * _D, _C * _D)], fv)
    pltpu.sync_copy(l_hbm.at[pl.ds(s * _C, _C)], lv)

    @pl.loop(0, _K)
    def _(k):
        accv[pl.ds(k * _D, 16)] = zeros16
        accv[pl.ds(k * _D + 16, 16)] = zeros16
        cnt2d[pl.ds(k * 16, 16)] = zeros16
        h2acc[pl.ds(k * 16, 16)] = zeros16

    h2acc[pl.ds(_K * 16, 16)] = zeros16

    @pl.loop(0, _G)
    def _(g):
        b = g * 16
        lblv = lv[pl.ds(b, 16)]
        plsc.addupdate_scatter(cnt2d, [lblv * 16 + iota16], ones16)
        for i in range(16):
            lbl = lblv[i]
            plsc.addupdate(accv.at[pl.ds(lbl * _D, 16)], fv[pl.ds((b + i) * _D, 16)])
            plsc.addupdate(accv.at[pl.ds(lbl * _D + 16, 16)],
                           fv[pl.ds((b + i) * _D + 16, 16)])

    pltpu.sync_copy(accv, psums_sp.at[s])
    pltpu.sync_copy(cnt2d, pcnts_sp.at[s])
    plsc.subcore_barrier()

    @pl.loop(0, _NT)
    def _(tt):
        pltpu.sync_copy(psums_sp.at[tt, pl.ds(2 * s * _D, 2 * _D)], pbuf.at[tt])
        pltpu.sync_copy(pcnts_sp.at[tt, pl.ds(2 * s * 16, 32)], pcbuf.at[tt])

    for ki in range(2):
        k = 2 * s + ki

        def _red(tt, carry, ki=ki):
            s0, s1, cv = carry
            return (s0 + pbuf[tt, ki * _D:ki * _D + 16],
                    s1 + pbuf[tt, ki * _D + 16:ki * _D + 32],
                    cv + pcbuf[tt, ki * 16:ki * 16 + 16])

        s0, s1, cv = pl.loop(0, _NT, init_carry=(zeros16, zeros16, zeros16))(_red)
        safe = jnp.maximum(jnp.sum(cv), 1.0)
        safev = _splat(safe)
        meansv[pl.ds(k * _D, 16)] = s0 / safev
        meansv[pl.ds(k * _D + 16, 16)] = s1 / safev
        scntv[pl.ds(k * 16, 16)] = safev

    pltpu.sync_copy(meansv.at[pl.ds(2 * s * _D, 2 * _D)],
                    means_sp.at[pl.ds(2 * s * _D, 2 * _D)])
    pltpu.sync_copy(scntv.at[pl.ds(2 * s * 16, 32)], cnts_sp.at[pl.ds(2 * s * 16, 32)])
    plsc.subcore_barrier()

    pltpu.sync_copy(means_sp, meansv)
    pltpu.sync_copy(cnts_sp, scntv)

    @pl.loop(0, _G)
    def _(g):
        b = g * 16
        lblv = lv[pl.ds(b, 16)]
        pbase = (iota16 + b) * _D
        mbase = lblv * _D
        d2 = zeros16
        for d in range(_D):
            xd = plsc.load_gather(fv, [pbase + d])
            md = plsc.load_gather(meansv, [mbase + d])
            t = xd - md
            d2 = d2 + t * t
        dist = _vsqrt(jnp.maximum(d2, _EPS))
        h = jnp.maximum(dist - _DELTA_VAR, 0.0)
        plsc.addupdate_scatter(h2acc, [lblv * 16 + iota16], h * h)

    cbaseA = iota16 * _D
    cbaseB = (iota16 + 16) * _D
    r0 = zeros16
    r1 = zeros16
    for d in range(_D):
        colA = plsc.load_gather(meansv, [cbaseA + d])
        colB = plsc.load_gather(meansv, [cbaseB + d])
        mtv[d, 0:16] = colA
        mtv[d, 16:32] = colB
        r0 = r0 + colA * colA
        r1 = r1 + colB * colB
    regn0 = _vsqrt(jnp.maximum(r0, _EPS))
    regn1 = _vsqrt(jnp.maximum(r1, _EPS))
    reg_sum = jnp.sum(regn0) + jnp.sum(regn1)

    dist_part = jnp.float32(0.0)
    for ki in range(2):
        k = 2 * s + ki
        mk0 = meansv[pl.ds(k * _D, 16)]
        mk1 = meansv[pl.ds(k * _D + 16, 16)]
        for jc in range(2):
            pd2 = zeros16
            for d in range(_D):
                mkd = mk0[d] if d < 16 else mk1[d - 16]
                t = mtv[d, jc * 16:jc * 16 + 16] - mkd
                pd2 = pd2 + t * t
            jv = iota16 + 16 * jc
            diag = jv == k
            pd = _vsqrt(jnp.where(diag, 1.0, jnp.maximum(pd2, _EPS)))
            h = jnp.maximum(2.0 * _DELTA_DIST - pd, 0.0)
            dist_part = dist_part + jnp.sum(jnp.where(diag, 0.0, h * h))

    h2acc[pl.ds(_K * 16, 16)] = jnp.where(iota16 == 0, dist_part, 0.0)
    pltpu.sync_copy(h2acc, ph2_sp.at[s])
    plsc.subcore_barrier()

    @pl.when(jnp.logical_and(s == 0, c == 0))
    def _():
        pltpu.sync_copy(ph2_sp, ph2buf)

        def _var(k, vacc):
            def _acc(tt, a16):
                return a16 + ph2buf[tt, pl.ds(k * 16, 16)]

            a16 = pl.loop(0, _NT, init_carry=zeros16)(_acc)
            return vacc + a16 / scntv[pl.ds(k * 16, 16)]

        var_vacc = pl.loop(0, _K, init_carry=zeros16)(_var)
        var_loss = jnp.sum(var_vacc) * (1.0 / _K)

        def _dist(tt, a16):
            return a16 + ph2buf[tt, pl.ds(_K * 16, 16)]

        dist_v = pl.loop(0, _NT, init_carry=zeros16)(_dist)
        dist_loss = jnp.sum(dist_v) * (1.0 / (_K * (_K - 1)))

        reg_loss = reg_sum * (1.0 / _K)
        loss = _ALPHA * var_loss + _BETA * dist_loss + _GAMMA * reg_loss

        res = jnp.where(iota16 == 0, loss,
                        jnp.where(iota16 == 1, var_loss,
                                  jnp.where(iota16 == 2, dist_loss,
                                            jnp.where(iota16 == 3, reg_loss, 0.0))))
        outv[:] = res
        pltpu.sync_copy(outv, out_hbm)


@jax.jit
def _run(features, labels):
    mesh = plsc.VectorSubcoreMesh(core_axis_name="c", subcore_axis_name="s",
                                  num_cores=2, num_subcores=16)
    f = pl.kernel(
        _sc_body,
        out_type=jax.ShapeDtypeStruct((16,), jnp.float32),
        mesh=mesh,
        compiler_params=pltpu.CompilerParams(needs_layout_passes=False),
        scratch_types=[
            pltpu.VMEM((_C * _D,), jnp.float32),
            pltpu.VMEM((_C,), jnp.int32),
            pltpu.VMEM((_K * _D,), jnp.float32),
            pltpu.VMEM((_K * 16,), jnp.float32),
            pltpu.VMEM((_K * 16 + 16,), jnp.float32),
            pltpu.VMEM((_K * _D,), jnp.float32),
            pltpu.VMEM((_K * 16,), jnp.float32),
            pltpu.VMEM((_D, _K), jnp.float32),
            pltpu.VMEM((_NT, 2 * _D), jnp.float32),
            pltpu.VMEM((_NT, 32), jnp.float32),
            pltpu.VMEM((_NT, _K * 16 + 16), jnp.float32),
            pltpu.VMEM((16,), jnp.float32),
            pltpu.VMEM_SHARED((_NT, _K * _D), jnp.float32),
            pltpu.VMEM_SHARED((_NT, _K * 16), jnp.float32),
            pltpu.VMEM_SHARED((_K * _D,), jnp.float32),
            pltpu.VMEM_SHARED((_K * 16,), jnp.float32),
            pltpu.VMEM_SHARED((_NT, _K * 16 + 16), jnp.float32),
        ],
    )
    return f(features.reshape(-1), labels)


def kernel(features, labels):
    out = _run(features, labels)
    return out[0], out[1], out[2], out[3]


# speedup vs baseline: 3.4620x; 1.1036x over previous
import jax
import jax.numpy as jnp
from jax import lax
from jax.experimental import pallas as pl
from jax.experimental.pallas import tpu as pltpu
from jax.experimental.pallas import tpu_sc as plsc

_K = 32
_D = 32
_N = 32768
_NT = 16
_C = _N // _NT
_G = _C // 16
_DELTA_VAR = 0.5
_DELTA_DIST = 1.5
_ALPHA = 0.1
_BETA = 1.0
_GAMMA = 0.001
_EPS = 1e-12


def _vsqrt(x):
    i = plsc.bitcast(x, jnp.int32)
    i = 0x5F3759DF - (i >> 1)
    r = plsc.bitcast(i, jnp.float32)
    for _ in range(3):
        r = r * (1.5 - 0.5 * x * r * r)
    return x * r


def _splat(s):
    return jnp.zeros((16,), jnp.float32) + s


def _sc_body(f_hbm, l_hbm, out_hbm,
             fv, lv, accv, cnt2d, h2acc, meansv, scntv, mtv,
             pbuf, pcbuf, ph2buf, outv,
             psums_sp, pcnts_sp, means_sp, cnts_sp, ph2_sp):
    s = lax.axis_index("s")
    c = lax.axis_index("c")
    iota16 = lax.iota(jnp.int32, 16)
    ones16 = jnp.ones((16,), jnp.float32)
    zeros16 = jnp.zeros((16,), jnp.float32)

    pltpu.sync_copy(f_hbm.at[pl.ds(s * _C * _D, _C * _D)], fv)
    pltpu.sync_copy(l_hbm.at[pl.ds(s * _C, _C)], lv)

    @pl.loop(0, _K)
    def _(k):
        accv[pl.ds(k * _D, 16)] = zeros16
        accv[pl.ds(k * _D + 16, 16)] = zeros16
        cnt2d[pl.ds(k * 16, 16)] = zeros16
        h2acc[pl.ds(k * 16, 16)] = zeros16

    h2acc[pl.ds(_K * 16, 16)] = zeros16

    def _p1(g, carry):
        cur, a0, a1 = carry
        b = g * 16
        lblv = lv[pl.ds(b, 16)]
        plsc.addupdate_scatter(cnt2d, [lblv * 16 + iota16], ones16)
        l0 = lblv[0]
        l15 = lblv[15]
        fast = jnp.logical_and(l0 == cur, l0 == l15)

        def _fast(cur, a0, a1):
            for i in range(16):
                a0 = a0 + fv[pl.ds((b + i) * _D, 16)]
                a1 = a1 + fv[pl.ds((b + i) * _D + 16, 16)]
            return cur, a0, a1

        def _slow(cur, a0, a1):
            plsc.addupdate(accv.at[pl.ds(cur * _D, 16)], a0)
            plsc.addupdate(accv.at[pl.ds(cur * _D + 16, 16)], a1)
            for i in range(16):
                lbl = lblv[i]
                plsc.addupdate(accv.at[pl.ds(lbl * _D, 16)],
                               fv[pl.ds((b + i) * _D, 16)])
                plsc.addupdate(accv.at[pl.ds(lbl * _D + 16, 16)],
                               fv[pl.ds((b + i) * _D + 16, 16)])
            return l15, zeros16, zeros16

        return lax.cond(fast, _fast, _slow, cur, a0, a1)

    cur, a0, a1 = pl.loop(0, _G, init_carry=(lv[pl.ds(0, 16)][0], zeros16, zeros16))(_p1)
    plsc.addupdate(accv.at[pl.ds(cur * _D, 16)], a0)
    plsc.addupdate(accv.at[pl.ds(cur * _D + 16, 16)], a1)

    pltpu.sync_copy(accv, psums_sp.at[s])
    pltpu.sync_copy(cnt2d, pcnts_sp.at[s])
    plsc.subcore_barrier()

    @pl.loop(0, _NT)
    def _(tt):
        pltpu.sync_copy(psums_sp.at[tt, pl.ds(2 * s * _D, 2 * _D)], pbuf.at[tt])
        pltpu.sync_copy(pcnts_sp.at[tt, pl.ds(2 * s * 16, 32)], pcbuf.at[tt])

    for ki in range(2):
        k = 2 * s + ki

        def _red(tt, carry, ki=ki):
            s0, s1, cv = carry
            return (s0 + pbuf[tt, ki * _D:ki * _D + 16],
                    s1 + pbuf[tt, ki * _D + 16:ki * _D + 32],
                    cv + pcbuf[tt, ki * 16:ki * 16 + 16])

        s0, s1, cv = pl.loop(0, _NT, init_carry=(zeros16, zeros16, zeros16))(_red)
        safe = jnp.maximum(jnp.sum(cv), 1.0)
        safev = _splat(safe)
        meansv[pl.ds(k * _D, 16)] = s0 / safev
        meansv[pl.ds(k * _D + 16, 16)] = s1 / safev
        scntv[pl.ds(k * 16, 16)] = safev

    pltpu.sync_copy(meansv.at[pl.ds(2 * s * _D, 2 * _D)],
                    means_sp.at[pl.ds(2 * s * _D, 2 * _D)])
    pltpu.sync_copy(scntv.at[pl.ds(2 * s * 16, 32)], cnts_sp.at[pl.ds(2 * s * 16, 32)])
    plsc.subcore_barrier()

    pltpu.sync_copy(means_sp, meansv)
    pltpu.sync_copy(cnts_sp, scntv)

    @pl.loop(0, _G)
    def _(g):
        b = g * 16
        lblv = lv[pl.ds(b, 16)]
        pbase = (iota16 + b) * _D
        mbase = lblv * _D
        d2 = zeros16
        for d in range(_D):
            xd = plsc.load_gather(fv, [pbase + d])
            md = plsc.load_gather(meansv, [mbase + d])
            t = xd - md
            d2 = d2 + t * t
        dist = _vsqrt(jnp.maximum(d2, _EPS))
        h = jnp.maximum(dist - _DELTA_VAR, 0.0)
        plsc.addupdate_scatter(h2acc, [lblv * 16 + iota16], h * h)

    cbaseA = iota16 * _D
    cbaseB = (iota16 + 16) * _D
    r0 = zeros16
    r1 = zeros16
    for d in range(_D):
        colA = plsc.load_gather(meansv, [cbaseA + d])
        colB = plsc.load_gather(meansv, [cbaseB + d])
        mtv[d, 0:16] = colA
        mtv[d, 16:32] = colB
        r0 = r0 + colA * colA
        r1 = r1 + colB * colB
    regn0 = _vsqrt(jnp.maximum(r0, _EPS))
    regn1 = _vsqrt(jnp.maximum(r1, _EPS))
    reg_sum = jnp.sum(regn0) + jnp.sum(regn1)

    dist_part = jnp.float32(0.0)
    for ki in range(2):
        k = 2 * s + ki
        mk0 = meansv[pl.ds(k * _D, 16)]
        mk1 = meansv[pl.ds(k * _D + 16, 16)]
        for jc in range(2):
            pd2 = zeros16
            for d in range(_D):
                mkd = mk0[d] if d < 16 else mk1[d - 16]
                t = mtv[d, jc * 16:jc * 16 + 16] - mkd
                pd2 = pd2 + t * t
            jv = iota16 + 16 * jc
            diag = jv == k
            pd = _vsqrt(jnp.where(diag, 1.0, jnp.maximum(pd2, _EPS)))
            h = jnp.maximum(2.0 * _DELTA_DIST - pd, 0.0)
            dist_part = dist_part + jnp.sum(jnp.where(diag, 0.0, h * h))

    h2acc[pl.ds(_K * 16, 16)] = jnp.where(iota16 == 0, dist_part, 0.0)
    pltpu.sync_copy(h2acc, ph2_sp.at[s])
    plsc.subcore_barrier()

    @pl.when(jnp.logical_and(s == 0, c == 0))
    def _():
        pltpu.sync_copy(ph2_sp, ph2buf)

        def _var(k, vacc):
            def _acc(tt, a16):
                return a16 + ph2buf[tt, pl.ds(k * 16, 16)]

            a16 = pl.loop(0, _NT, init_carry=zeros16)(_acc)
            return vacc + a16 / scntv[pl.ds(k * 16, 16)]

        var_vacc = pl.loop(0, _K, init_carry=zeros16)(_var)
        var_loss = jnp.sum(var_vacc) * (1.0 / _K)

        def _dist(tt, a16):
            return a16 + ph2buf[tt, pl.ds(_K * 16, 16)]

        dist_v = pl.loop(0, _NT, init_carry=zeros16)(_dist)
        dist_loss = jnp.sum(dist_v) * (1.0 / (_K * (_K - 1)))

        reg_loss = reg_sum * (1.0 / _K)
        loss = _ALPHA * var_loss + _BETA * dist_loss + _GAMMA * reg_loss

        res = jnp.where(iota16 == 0, loss,
                        jnp.where(iota16 == 1, var_loss,
                                  jnp.where(iota16 == 2, dist_loss,
                                            jnp.where(iota16 == 3, reg_loss, 0.0))))
        outv[:] = res
        pltpu.sync_copy(outv, out_hbm)


@jax.jit
def _run(features, labels):
    mesh = plsc.VectorSubcoreMesh(core_axis_name="c", subcore_axis_name="s",
                                  num_cores=2, num_subcores=16)
    f = pl.kernel(
        _sc_body,
        out_type=jax.ShapeDtypeStruct((16,), jnp.float32),
        mesh=mesh,
        compiler_params=pltpu.CompilerParams(needs_layout_passes=False),
        scratch_types=[
            pltpu.VMEM((_C * _D,), jnp.float32),
            pltpu.VMEM((_C,), jnp.int32),
            pltpu.VMEM((_K * _D,), jnp.float32),
            pltpu.VMEM((_K * 16,), jnp.float32),
            pltpu.VMEM((_K * 16 + 16,), jnp.float32),
            pltpu.VMEM((_K * _D,), jnp.float32),
            pltpu.VMEM((_K * 16,), jnp.float32),
            pltpu.VMEM((_D, _K), jnp.float32),
            pltpu.VMEM((_NT, 2 * _D), jnp.float32),
            pltpu.VMEM((_NT, 32), jnp.float32),
            pltpu.VMEM((_NT, _K * 16 + 16), jnp.float32),
            pltpu.VMEM((16,), jnp.float32),
            pltpu.VMEM_SHARED((_NT, _K * _D), jnp.float32),
            pltpu.VMEM_SHARED((_NT, _K * 16), jnp.float32),
            pltpu.VMEM_SHARED((_K * _D,), jnp.float32),
            pltpu.VMEM_SHARED((_K * 16,), jnp.float32),
            pltpu.VMEM_SHARED((_NT, _K * 16 + 16), jnp.float32),
        ],
    )
    return f(features.reshape(-1), labels)


def kernel(features, labels):
    out = _run(features, labels)
    return out[0], out[1], out[2], out[3]


# speedup vs baseline: 3.5734x; 1.0322x over previous
import jax
import jax.numpy as jnp
from jax import lax
from jax.experimental import pallas as pl
from jax.experimental.pallas import tpu as pltpu
from jax.experimental.pallas import tpu_sc as plsc

_K = 32
_D = 32
_N = 32768
_NT = 16
_C = _N // _NT
_G = _C // 16
_DELTA_VAR = 0.5
_DELTA_DIST = 1.5
_ALPHA = 0.1
_BETA = 1.0
_GAMMA = 0.001
_EPS = 1e-12


def _vsqrt(x):
    i = plsc.bitcast(x, jnp.int32)
    i = 0x5F3759DF - (i >> 1)
    r = plsc.bitcast(i, jnp.float32)
    for _ in range(3):
        r = r * (1.5 - 0.5 * x * r * r)
    return x * r


def _splat(s):
    return jnp.zeros((16,), jnp.float32) + s


def _sc_body(f_hbm, l_hbm, out_hbm,
             fv, lv, accv, cnt2d, h2acc, meansv, scntv, mtv,
             pbuf, pcbuf, ph2buf, outv,
             psums_sp, pcnts_sp, means_sp, cnts_sp, ph2_sp):
    s = lax.axis_index("s")
    c = lax.axis_index("c")
    iota16 = lax.iota(jnp.int32, 16)
    ones16 = jnp.ones((16,), jnp.float32)
    zeros16 = jnp.zeros((16,), jnp.float32)

    pltpu.sync_copy(f_hbm.at[pl.ds(s * _C * _D, _C * _D)], fv)
    pltpu.sync_copy(l_hbm.at[pl.ds(s * _C, _C)], lv)

    @pl.loop(0, _K)
    def _(k):
        accv[pl.ds(k * _D, 16)] = zeros16
        accv[pl.ds(k * _D + 16, 16)] = zeros16
        cnt2d[pl.ds(k * 16, 16)] = zeros16
        h2acc[pl.ds(k * 16, 16)] = zeros16

    h2acc[pl.ds(_K * 16, 16)] = zeros16

    def _p1(g, carry):
        cur, a0, a1 = carry
        b = g * 16
        lblv = lv[pl.ds(b, 16)]
        plsc.addupdate_scatter(cnt2d, [lblv * 16 + iota16], ones16)
        l0 = lblv[0]
        l15 = lblv[15]
        fast = jnp.logical_and(l0 == cur, l0 == l15)

        def _fast(cur, a0, a1):
            for i in range(16):
                a0 = a0 + fv[pl.ds((b + i) * _D, 16)]
                a1 = a1 + fv[pl.ds((b + i) * _D + 16, 16)]
            return cur, a0, a1

        def _slow(cur, a0, a1):
            plsc.addupdate(accv.at[pl.ds(cur * _D, 16)], a0)
            plsc.addupdate(accv.at[pl.ds(cur * _D + 16, 16)], a1)
            for i in range(16):
                lbl = lblv[i]
                plsc.addupdate(accv.at[pl.ds(lbl * _D, 16)],
                               fv[pl.ds((b + i) * _D, 16)])
                plsc.addupdate(accv.at[pl.ds(lbl * _D + 16, 16)],
                               fv[pl.ds((b + i) * _D + 16, 16)])
            return l15, zeros16, zeros16

        return lax.cond(fast, _fast, _slow, cur, a0, a1)

    cur, a0, a1 = pl.loop(0, _G, init_carry=(lv[pl.ds(0, 16)][0], zeros16, zeros16))(_p1)
    plsc.addupdate(accv.at[pl.ds(cur * _D, 16)], a0)
    plsc.addupdate(accv.at[pl.ds(cur * _D + 16, 16)], a1)

    pltpu.sync_copy(accv, psums_sp.at[s])
    pltpu.sync_copy(cnt2d, pcnts_sp.at[s])
    plsc.subcore_barrier()

    @pl.loop(0, _NT)
    def _(tt):
        pltpu.sync_copy(psums_sp.at[tt, pl.ds(2 * s * _D, 2 * _D)], pbuf.at[tt])
        pltpu.sync_copy(pcnts_sp.at[tt, pl.ds(2 * s * 16, 32)], pcbuf.at[tt])

    for ki in range(2):
        k = 2 * s + ki

        def _red(tt, carry, ki=ki):
            s0, s1, cv = carry
            return (s0 + pbuf[tt, ki * _D:ki * _D + 16],
                    s1 + pbuf[tt, ki * _D + 16:ki * _D + 32],
                    cv + pcbuf[tt, ki * 16:ki * 16 + 16])

        s0, s1, cv = pl.loop(0, _NT, init_carry=(zeros16, zeros16, zeros16))(_red)
        safe = jnp.maximum(jnp.sum(cv), 1.0)
        safev = _splat(safe)
        meansv[pl.ds(k * _D, 16)] = s0 / safev
        meansv[pl.ds(k * _D + 16, 16)] = s1 / safev
        scntv[pl.ds(k * 16, 16)] = safev

    pltpu.sync_copy(meansv.at[pl.ds(2 * s * _D, 2 * _D)],
                    means_sp.at[pl.ds(2 * s * _D, 2 * _D)])
    pltpu.sync_copy(scntv.at[pl.ds(2 * s * 16, 32)], cnts_sp.at[pl.ds(2 * s * 16, 32)])
    plsc.subcore_barrier()

    pltpu.sync_copy(means_sp, meansv)
    pltpu.sync_copy(cnts_sp, scntv)

    @pl.loop(0, _G)
    def _(g):
        b = g * 16
        lblv = lv[pl.ds(b, 16)]
        pbase = (iota16 + b) * _D
        l0 = lblv[0]
        uniform = l0 == lblv[15]

        def _fast():
            mk0 = meansv[pl.ds(l0 * _D, 16)]
            mk1 = meansv[pl.ds(l0 * _D + 16, 16)]
            d2 = zeros16
            for d in range(_D):
                xd = plsc.load_gather(fv, [pbase + d])
                mkd = mk0[d] if d < 16 else mk1[d - 16]
                t = xd - mkd
                d2 = d2 + t * t
            return d2

        def _slow():
            mbase = lblv * _D
            d2 = zeros16
            for d in range(_D):
                xd = plsc.load_gather(fv, [pbase + d])
                md = plsc.load_gather(meansv, [mbase + d])
                t = xd - md
                d2 = d2 + t * t
            return d2

        d2 = lax.cond(uniform, _fast, _slow)
        dist = _vsqrt(jnp.maximum(d2, _EPS))
        h = jnp.maximum(dist - _DELTA_VAR, 0.0)
        plsc.addupdate_scatter(h2acc, [lblv * 16 + iota16], h * h)

    cbaseA = iota16 * _D
    cbaseB = (iota16 + 16) * _D
    r0 = zeros16
    r1 = zeros16
    for d in range(_D):
        colA = plsc.load_gather(meansv, [cbaseA + d])
        colB = plsc.load_gather(meansv, [cbaseB + d])
        mtv[d, 0:16] = colA
        mtv[d, 16:32] = colB
        r0 = r0 + colA * colA
        r1 = r1 + colB * colB
    regn0 = _vsqrt(jnp.maximum(r0, _EPS))
    regn1 = _vsqrt(jnp.maximum(r1, _EPS))
    reg_sum = jnp.sum(regn0) + jnp.sum(regn1)

    dist_part = jnp.float32(0.0)
    for ki in range(2):
        k = 2 * s + ki
        mk0 = meansv[pl.ds(k * _D, 16)]
        mk1 = meansv[pl.ds(k * _D + 16, 16)]
        for jc in range(2):
            pd2 = zeros16
            for d in range(_D):
                mkd = mk0[d] if d < 16 else mk1[d - 16]
                t = mtv[d, jc * 16:jc * 16 + 16] - mkd
                pd2 = pd2 + t * t
            jv = iota16 + 16 * jc
            diag = jv == k
            pd = _vsqrt(jnp.where(diag, 1.0, jnp.maximum(pd2, _EPS)))
            h = jnp.maximum(2.0 * _DELTA_DIST - pd, 0.0)
            dist_part = dist_part + jnp.sum(jnp.where(diag, 0.0, h * h))

    h2acc[pl.ds(_K * 16, 16)] = jnp.where(iota16 == 0, dist_part, 0.0)
    pltpu.sync_copy(h2acc, ph2_sp.at[s])
    plsc.subcore_barrier()

    @pl.when(jnp.logical_and(s == 0, c == 0))
    def _():
        pltpu.sync_copy(ph2_sp, ph2buf)

        def _var(k, vacc):
            def _acc(tt, a16):
                return a16 + ph2buf[tt, pl.ds(k * 16, 16)]

            a16 = pl.loop(0, _NT, init_carry=zeros16)(_acc)
            return vacc + a16 / scntv[pl.ds(k * 16, 16)]

        var_vacc = pl.loop(0, _K, init_carry=zeros16)(_var)
        var_loss = jnp.sum(var_vacc) * (1.0 / _K)

        def _dist(tt, a16):
            return a16 + ph2buf[tt, pl.ds(_K * 16, 16)]

        dist_v = pl.loop(0, _NT, init_carry=zeros16)(_dist)
        dist_loss = jnp.sum(dist_v) * (1.0 / (_K * (_K - 1)))

        reg_loss = reg_sum * (1.0 / _K)
        loss = _ALPHA * var_loss + _BETA * dist_loss + _GAMMA * reg_loss

        res = jnp.where(iota16 == 0, loss,
                        jnp.where(iota16 == 1, var_loss,
                                  jnp.where(iota16 == 2, dist_loss,
                                            jnp.where(iota16 == 3, reg_loss, 0.0))))
        outv[:] = res
        pltpu.sync_copy(outv, out_hbm)


@jax.jit
def _run(features, labels):
    mesh = plsc.VectorSubcoreMesh(core_axis_name="c", subcore_axis_name="s",
                                  num_cores=2, num_subcores=16)
    f = pl.kernel(
        _sc_body,
        out_type=jax.ShapeDtypeStruct((16,), jnp.float32),
        mesh=mesh,
        compiler_params=pltpu.CompilerParams(needs_layout_passes=False),
        scratch_types=[
            pltpu.VMEM((_C * _D,), jnp.float32),
            pltpu.VMEM((_C,), jnp.int32),
            pltpu.VMEM((_K * _D,), jnp.float32),
            pltpu.VMEM((_K * 16,), jnp.float32),
            pltpu.VMEM((_K * 16 + 16,), jnp.float32),
            pltpu.VMEM((_K * _D,), jnp.float32),
            pltpu.VMEM((_K * 16,), jnp.float32),
            pltpu.VMEM((_D, _K), jnp.float32),
            pltpu.VMEM((_NT, 2 * _D), jnp.float32),
            pltpu.VMEM((_NT, 32), jnp.float32),
            pltpu.VMEM((_NT, _K * 16 + 16), jnp.float32),
            pltpu.VMEM((16,), jnp.float32),
            pltpu.VMEM_SHARED((_NT, _K * _D), jnp.float32),
            pltpu.VMEM_SHARED((_NT, _K * 16), jnp.float32),
            pltpu.VMEM_SHARED((_K * _D,), jnp.float32),
            pltpu.VMEM_SHARED((_K * 16,), jnp.float32),
            pltpu.VMEM_SHARED((_NT, _K * 16 + 16), jnp.float32),
        ],
    )
    return f(features.reshape(-1), labels)


def kernel(features, labels):
    out = _run(features, labels)
    return out[0], out[1], out[2], out[3]
